# Initial kernel scaffold; baseline (speedup 1.0000x reference)
import jax
import jax.numpy as jnp
from jax.experimental import pallas as pl


def kernel(x, edge_index, Wl1, bl1, Wr1, Wl2, bl2, Wr2, Wl3, bl3, Wr3):
    raise NotImplementedError("write your pallas kernel here")


# speedup vs baseline: 6.1348x; 6.1348x over previous
import functools

import jax
import jax.numpy as jnp
from jax import lax
from jax.experimental import pallas as pl
from jax.experimental.pallas import tpu as pltpu
from jax.experimental.pallas import tpu_sc as plsc

N = 10000
E = 320000
D = 128

NC = 2
NS = 16
NW = NC * NS
CH = 128
EPT = 10240
EPAD = EPT * NW
NCHUNK = EPT // CH
NP = 10240
RPT = NP // NS
CW = 16

@functools.cache
def _sc_kernels():
    mesh = plsc.VectorSubcoreMesh(core_axis_name="c", subcore_axis_name="s",
                                  num_cores=NC, num_subcores=NS)

    @functools.partial(
        pl.kernel,
        out_type=jax.ShapeDtypeStruct((NC, NP, D), jnp.float32),
        mesh=mesh,
        scratch_types=[
            pltpu.VMEM((CH,), jnp.int32),
            pltpu.VMEM((CH,), jnp.int32),
            pltpu.VMEM((CH, D), jnp.float32),
            pltpu.VMEM_SHARED((NP, D), jnp.float32),
            pltpu.SemaphoreType.DMA,
        ],
    )
    def sc_aggregate(h_hbm, src_hbm, dst_hbm, zrows_hbm, out_hbm,
                     sidx, didx, rows, acc, sem):
        cid = lax.axis_index("c")
        sid = lax.axis_index("s")
        wid = sid * NC + cid
        row0 = pl.multiple_of(sid * RPT, 8)
        pltpu.sync_copy(zrows_hbm, acc.at[pl.ds(row0, RPT)])
        plsc.subcore_barrier()
        base = wid * EPT

        def body(g, carry):
            off = pl.multiple_of(base + g * CH, 8)
            pltpu.sync_copy(src_hbm.at[pl.ds(off, CH)], sidx)
            pltpu.sync_copy(dst_hbm.at[pl.ds(off, CH)], didx)
            pltpu.async_copy(h_hbm.at[sidx], rows, sem).wait()
            pltpu.sync_copy(rows, acc.at[didx], add=True)
            return carry

        lax.fori_loop(0, NCHUNK, body, 0)
        plsc.subcore_barrier()
        pltpu.sync_copy(acc.at[pl.ds(row0, RPT)],
                        out_hbm.at[cid, pl.ds(row0, RPT)])

    @functools.partial(
        pl.kernel,
        out_type=jax.ShapeDtypeStruct((NC * NP,), jnp.float32),
        mesh=mesh,
        scratch_types=[
            pltpu.VMEM((CH,), jnp.int32),
            pltpu.VMEM((CH,), jnp.float32),
            pltpu.VMEM_SHARED((NP,), jnp.float32),
        ],
    )
    def sc_count(dst_hbm, ones_hbm, zcnt_hbm, out_hbm, didx, ones, cnt):
        cid = lax.axis_index("c")
        sid = lax.axis_index("s")
        wid = sid * NC + cid
        row0 = pl.multiple_of(sid * RPT, 8)
        pltpu.sync_copy(zcnt_hbm, cnt.at[pl.ds(row0, RPT)])
        pltpu.sync_copy(ones_hbm, ones)
        plsc.subcore_barrier()
        base = wid * EPT

        def body(g, carry):
            off = pl.multiple_of(base + g * CH, 8)
            pltpu.sync_copy(dst_hbm.at[pl.ds(off, CH)], didx)
            pltpu.sync_copy(ones, cnt.at[didx], add=True)
            return carry

        lax.fori_loop(0, NCHUNK, body, 0)
        plsc.subcore_barrier()
        out0 = pl.multiple_of(cid * NP + sid * RPT, 8)
        pltpu.sync_copy(cnt.at[pl.ds(row0, RPT)],
                        out_hbm.at[pl.ds(out0, RPT)])

    return sc_aggregate, sc_count


BN = 1024


def _mm_body(relu, p_ref, cnt_ref, h_ref, wlt_ref, bl_ref, wrt_ref, o_ref):
    cnt = cnt_ref[0] + cnt_ref[1]
    inv = 1.0 / jnp.maximum(cnt, 1.0)
    mean = (p_ref[0] + p_ref[1]) * inv[:, None]
    out = (jnp.dot(mean, wlt_ref[...], preferred_element_type=jnp.float32)
           + bl_ref[...]
           + jnp.dot(h_ref[...], wrt_ref[...], preferred_element_type=jnp.float32))
    if relu:
        out = jnp.maximum(out, 0.0)
    o_ref[...] = out


def _tc_sage_mm(relu, p, cnt, h, wlt, bl, wrt):
    return pl.pallas_call(
        functools.partial(_mm_body, relu),
        grid=(NP // BN,),
        in_specs=[
            pl.BlockSpec((NC, BN, D), lambda i: (0, i, 0)),
            pl.BlockSpec((NC, BN), lambda i: (0, i)),
            pl.BlockSpec((BN, D), lambda i: (i, 0)),
            pl.BlockSpec((D, D), lambda i: (0, 0)),
            pl.BlockSpec((1, D), lambda i: (0, 0)),
            pl.BlockSpec((D, D), lambda i: (0, 0)),
        ],
        out_specs=pl.BlockSpec((BN, D), lambda i: (i, 0)),
        out_shape=jax.ShapeDtypeStruct((NP, D), jnp.float32),
    )(p, cnt, h, wlt, bl, wrt)


def kernel(x, edge_index, Wl1, bl1, Wr1, Wl2, bl2, Wr2, Wl3, bl3, Wr3):
    src = edge_index[0].astype(jnp.int32)
    dst = edge_index[1].astype(jnp.int32)
    pad = EPAD - E
    ar = jnp.arange(pad, dtype=jnp.int32)
    srcp = jnp.concatenate([src, (ar * 97) % N])
    dstp = jnp.concatenate([dst, N + ar % (NP - N)])

    zrows = jnp.zeros((RPT, D), jnp.float32)
    zcnt = jnp.zeros((RPT,), jnp.float32)
    ones = jnp.ones((CH,), jnp.float32)
    xp = jnp.concatenate([x, jnp.zeros((NP - N, D), jnp.float32)])

    sc_aggregate, sc_count = _sc_kernels()
    cnt = sc_count(dstp, ones, zcnt).reshape(NC, NP)

    h = xp
    for (Wl, bl, Wr, relu) in ((Wl1, bl1, Wr1, True),
                               (Wl2, bl2, Wr2, True),
                               (Wl3, bl3, Wr3, False)):
        p = sc_aggregate(h, srcp, dstp, zrows)
        h = _tc_sage_mm(relu, p, cnt, h, Wl.T, bl.reshape(1, D), Wr.T)
    return h[:N]


# speedup vs baseline: 10.6037x; 1.7285x over previous
import functools

import jax
import jax.numpy as jnp
from jax import lax
from jax.experimental import pallas as pl
from jax.experimental.pallas import tpu as pltpu
from jax.experimental.pallas import tpu_sc as plsc

N = 10000
E = 320000
D = 128

NC = 2
NS = 16
NW = NC * NS
CH = 128
EPT = 10240
EPAD = EPT * NW
NCHUNK = EPT // CH
NP = 10240
RPT = NP // NS
CW = 16

@functools.cache
def _sc_kernels():
    mesh = plsc.VectorSubcoreMesh(core_axis_name="c", subcore_axis_name="s",
                                  num_cores=NC, num_subcores=NS)

    @functools.partial(
        pl.kernel,
        out_type=jax.ShapeDtypeStruct((NC, NP, D), jnp.float32),
        mesh=mesh,
        scratch_types=[
            pltpu.VMEM((2, CH), jnp.int32),
            pltpu.VMEM((2, CH), jnp.int32),
            pltpu.VMEM((CH, D), jnp.float32),
            pltpu.VMEM((CH, D), jnp.float32),
            pltpu.VMEM_SHARED((NP, D), jnp.float32),
            pltpu.SemaphoreType.DMA,
            pltpu.SemaphoreType.DMA,
        ],
    )
    def sc_aggregate(h_hbm, eidx_hbm, zrows_hbm, out_hbm,
                     idx0, idx1, rows0, rows1, acc, sem0, sem1):
        cid = lax.axis_index("c")
        sid = lax.axis_index("s")
        wid = sid * NC + cid
        row0 = pl.multiple_of(sid * RPT, 8)
        pltpu.sync_copy(zrows_hbm, acc.at[pl.ds(row0, RPT)])
        plsc.subcore_barrier()
        base = wid * NCHUNK
        idx = (idx0, idx1)
        rows = (rows0, rows1)
        sem = (sem0, sem1)

        def load(g, b):
            pltpu.sync_copy(eidx_hbm.at[base + g], idx[b])

        def gather(g, b):
            del g
            pltpu.async_copy(h_hbm.at[idx[b].at[0]], rows[b], sem[b])

        def scatter(g, b):
            del g
            pltpu.sync_copy(rows[b], acc.at[idx[b].at[1]], add=True)

        load(0, 0)
        gather(0, 0)

        def body(gg, carry):
            g = gg * 2
            load(g + 1, 1)
            gather(g + 1, 1)
            pltpu.make_async_copy(h_hbm.at[idx[0].at[0]], rows[0], sem[0]).wait()
            scatter(g, 0)
            load(g + 2, 0)
            gather(g + 2, 0)
            pltpu.make_async_copy(h_hbm.at[idx[1].at[0]], rows[1], sem[1]).wait()
            scatter(g + 1, 1)
            return carry

        lax.fori_loop(0, NCHUNK // 2 - 1, body, 0)
        g = NCHUNK - 2
        load(g + 1, 1)
        gather(g + 1, 1)
        pltpu.make_async_copy(h_hbm.at[idx[0].at[0]], rows[0], sem[0]).wait()
        scatter(g, 0)
        pltpu.make_async_copy(h_hbm.at[idx[1].at[0]], rows[1], sem[1]).wait()
        scatter(g + 1, 1)

        plsc.subcore_barrier()
        pltpu.sync_copy(acc.at[pl.ds(row0, RPT)],
                        out_hbm.at[cid, pl.ds(row0, RPT)])

    @functools.partial(
        pl.kernel,
        out_type=jax.ShapeDtypeStruct((NC * NP,), jnp.float32),
        mesh=mesh,
        scratch_types=[
            pltpu.VMEM((CH,), jnp.int32),
            pltpu.VMEM((CH,), jnp.float32),
            pltpu.VMEM_SHARED((NP,), jnp.float32),
        ],
    )
    def sc_count(dst_hbm, ones_hbm, zcnt_hbm, out_hbm, didx, ones, cnt):
        cid = lax.axis_index("c")
        sid = lax.axis_index("s")
        wid = sid * NC + cid
        row0 = pl.multiple_of(sid * RPT, 8)
        pltpu.sync_copy(zcnt_hbm, cnt.at[pl.ds(row0, RPT)])
        pltpu.sync_copy(ones_hbm, ones)
        plsc.subcore_barrier()
        base = wid * EPT

        def body(g, carry):
            off = pl.multiple_of(base + g * CH, 8)
            pltpu.sync_copy(dst_hbm.at[pl.ds(off, CH)], didx)
            pltpu.sync_copy(ones, cnt.at[didx], add=True)
            return carry

        lax.fori_loop(0, NCHUNK, body, 0)
        plsc.subcore_barrier()
        out0 = pl.multiple_of(cid * NP + sid * RPT, 8)
        pltpu.sync_copy(cnt.at[pl.ds(row0, RPT)],
                        out_hbm.at[pl.ds(out0, RPT)])

    return sc_aggregate, sc_count


BN = 1024


def _mm_body(relu, p_ref, cnt_ref, h_ref, wlt_ref, bl_ref, wrt_ref, o_ref):
    cnt = cnt_ref[0] + cnt_ref[1]
    inv = 1.0 / jnp.maximum(cnt, 1.0)
    mean = (p_ref[0] + p_ref[1]) * inv[:, None]
    out = (jnp.dot(mean, wlt_ref[...], preferred_element_type=jnp.float32)
           + bl_ref[...]
           + jnp.dot(h_ref[...], wrt_ref[...], preferred_element_type=jnp.float32))
    if relu:
        out = jnp.maximum(out, 0.0)
    o_ref[...] = out


def _tc_sage_mm(relu, p, cnt, h, wlt, bl, wrt):
    return pl.pallas_call(
        functools.partial(_mm_body, relu),
        grid=(NP // BN,),
        in_specs=[
            pl.BlockSpec((NC, BN, D), lambda i: (0, i, 0)),
            pl.BlockSpec((NC, BN), lambda i: (0, i)),
            pl.BlockSpec((BN, D), lambda i: (i, 0)),
            pl.BlockSpec((D, D), lambda i: (0, 0)),
            pl.BlockSpec((1, D), lambda i: (0, 0)),
            pl.BlockSpec((D, D), lambda i: (0, 0)),
        ],
        out_specs=pl.BlockSpec((BN, D), lambda i: (i, 0)),
        out_shape=jax.ShapeDtypeStruct((NP, D), jnp.float32),
    )(p, cnt, h, wlt, bl, wrt)


def kernel(x, edge_index, Wl1, bl1, Wr1, Wl2, bl2, Wr2, Wl3, bl3, Wr3):
    src = edge_index[0].astype(jnp.int32)
    dst = edge_index[1].astype(jnp.int32)
    pad = EPAD - E
    ar = jnp.arange(pad, dtype=jnp.int32)
    srcp = jnp.concatenate([src, (ar * 97) % N])
    dstp = jnp.concatenate([dst, N + ar % (NP - N)])

    zrows = jnp.zeros((RPT, D), jnp.float32)
    zcnt = jnp.zeros((RPT,), jnp.float32)
    ones = jnp.ones((CH,), jnp.float32)
    xp = jnp.concatenate([x, jnp.zeros((NP - N, D), jnp.float32)])

    eidx = jnp.stack([srcp.reshape(-1, CH), dstp.reshape(-1, CH)], axis=1)

    sc_aggregate, sc_count = _sc_kernels()
    cnt = sc_count(dstp, ones, zcnt).reshape(NC, NP)

    h = xp
    for (Wl, bl, Wr, relu) in ((Wl1, bl1, Wr1, True),
                               (Wl2, bl2, Wr2, True),
                               (Wl3, bl3, Wr3, False)):
        p = sc_aggregate(h, eidx, zrows)
        h = _tc_sage_mm(relu, p, cnt, h, Wl.T, bl.reshape(1, D), Wr.T)
    return h[:N]


# speedup vs baseline: 11.6992x; 1.1033x over previous
import functools

import jax
import jax.numpy as jnp
from jax import lax
from jax.experimental import pallas as pl
from jax.experimental.pallas import tpu as pltpu
from jax.experimental.pallas import tpu_sc as plsc

N = 10000
E = 320000
D = 128

NC = 2
NS = 16
NW = NC * NS
CH = 128
EPT = 10240
EPAD = EPT * NW
NCHUNK = EPT // CH
NP = 10240
RPT = NP // NS
CW = 16

@functools.cache
def _sc_kernels():
    mesh = plsc.VectorSubcoreMesh(core_axis_name="c", subcore_axis_name="s",
                                  num_cores=NC, num_subcores=NS)

    def _agg_body(with_count, h_hbm, eidx_hbm, zrows_hbm, ones_hbm, zcnt_hbm,
                  out_hbm, cnt_out_hbm,
                  idx0, idx1, rows0, rows1, ones_v, acc, cnt, sem0, sem1, semc):
        cid = lax.axis_index("c")
        sid = lax.axis_index("s")
        wid = sid * NC + cid
        row0 = pl.multiple_of(sid * RPT, 8)
        pltpu.sync_copy(zrows_hbm, acc.at[pl.ds(row0, RPT)])
        if with_count:
            pltpu.sync_copy(zcnt_hbm, cnt.at[pl.ds(row0, RPT)])
            pltpu.sync_copy(ones_hbm, ones_v)
        plsc.subcore_barrier()
        base = wid * NCHUNK
        idx = (idx0, idx1)
        rows = (rows0, rows1)
        sem = (sem0, sem1)

        def load(g, b):
            pltpu.sync_copy(eidx_hbm.at[base + g], idx[b])

        def gather(g, b):
            del g
            pltpu.async_copy(h_hbm.at[idx[b].at[0]], rows[b], sem[b])

        def scatter(g, b):
            del g
            if with_count:
                pltpu.async_copy(ones_v, cnt.at[idx[b].at[1]], semc, add=True)
            pltpu.sync_copy(rows[b], acc.at[idx[b].at[1]], add=True)
            if with_count:
                pltpu.make_async_copy(ones_v, cnt.at[idx[b].at[1]], semc).wait()

        load(0, 0)
        gather(0, 0)

        def body(gg, carry):
            g = gg * 2
            load(g + 1, 1)
            gather(g + 1, 1)
            pltpu.make_async_copy(h_hbm.at[idx[0].at[0]], rows[0], sem[0]).wait()
            scatter(g, 0)
            load(g + 2, 0)
            gather(g + 2, 0)
            pltpu.make_async_copy(h_hbm.at[idx[1].at[0]], rows[1], sem[1]).wait()
            scatter(g + 1, 1)
            return carry

        lax.fori_loop(0, NCHUNK // 2 - 1, body, 0)
        g = NCHUNK - 2
        load(g + 1, 1)
        gather(g + 1, 1)
        pltpu.make_async_copy(h_hbm.at[idx[0].at[0]], rows[0], sem[0]).wait()
        scatter(g, 0)
        pltpu.make_async_copy(h_hbm.at[idx[1].at[0]], rows[1], sem[1]).wait()
        scatter(g + 1, 1)

        plsc.subcore_barrier()
        pltpu.sync_copy(acc.at[pl.ds(row0, RPT)],
                        out_hbm.at[cid, pl.ds(row0, RPT)])
        if with_count:
            cnt0 = pl.multiple_of(cid * NP + sid * RPT, 8)
            pltpu.sync_copy(cnt.at[pl.ds(row0, RPT)],
                            cnt_out_hbm.at[pl.ds(cnt0, RPT)])

    _agg_scratch = [
        pltpu.VMEM((2, CH), jnp.int32),
        pltpu.VMEM((2, CH), jnp.int32),
        pltpu.VMEM((CH, D), jnp.float32),
        pltpu.VMEM((CH, D), jnp.float32),
        pltpu.VMEM((CH,), jnp.float32),
        pltpu.VMEM_SHARED((NP, D), jnp.float32),
        pltpu.VMEM_SHARED((NP,), jnp.float32),
        pltpu.SemaphoreType.DMA,
        pltpu.SemaphoreType.DMA,
        pltpu.SemaphoreType.DMA,
    ]

    @functools.partial(
        pl.kernel,
        out_type=(jax.ShapeDtypeStruct((NC, NP, D), jnp.float32),
                  jax.ShapeDtypeStruct((NC * NP,), jnp.float32)),
        mesh=mesh,
        scratch_types=_agg_scratch,
    )
    def sc_aggregate_cnt(*args):
        _agg_body(True, *args)

    @functools.partial(
        pl.kernel,
        out_type=(jax.ShapeDtypeStruct((NC, NP, D), jnp.float32),
                  jax.ShapeDtypeStruct((NC * NP,), jnp.float32)),
        mesh=mesh,
        scratch_types=_agg_scratch,
    )
    def sc_aggregate(*args):
        _agg_body(False, *args)

    return sc_aggregate, sc_aggregate_cnt


BN = 1024


def _mm_body(relu, p_ref, cnt_ref, h_ref, wlt_ref, bl_ref, wrt_ref, o_ref):
    cnt = cnt_ref[0] + cnt_ref[1]
    inv = 1.0 / jnp.maximum(cnt, 1.0)
    mean = (p_ref[0] + p_ref[1]) * inv[:, None]
    out = (jnp.dot(mean, wlt_ref[...], preferred_element_type=jnp.float32)
           + bl_ref[...]
           + jnp.dot(h_ref[...], wrt_ref[...], preferred_element_type=jnp.float32))
    if relu:
        out = jnp.maximum(out, 0.0)
    o_ref[...] = out


def _tc_sage_mm(relu, p, cnt, h, wlt, bl, wrt):
    return pl.pallas_call(
        functools.partial(_mm_body, relu),
        grid=(NP // BN,),
        in_specs=[
            pl.BlockSpec((NC, BN, D), lambda i: (0, i, 0)),
            pl.BlockSpec((NC, BN), lambda i: (0, i)),
            pl.BlockSpec((BN, D), lambda i: (i, 0)),
            pl.BlockSpec((D, D), lambda i: (0, 0)),
            pl.BlockSpec((1, D), lambda i: (0, 0)),
            pl.BlockSpec((D, D), lambda i: (0, 0)),
        ],
        out_specs=pl.BlockSpec((BN, D), lambda i: (i, 0)),
        out_shape=jax.ShapeDtypeStruct((NP, D), jnp.float32),
    )(p, cnt, h, wlt, bl, wrt)


def kernel(x, edge_index, Wl1, bl1, Wr1, Wl2, bl2, Wr2, Wl3, bl3, Wr3):
    src = edge_index[0].astype(jnp.int32)
    dst = edge_index[1].astype(jnp.int32)
    pad = EPAD - E
    ar = jnp.arange(pad, dtype=jnp.int32)
    srcp = jnp.concatenate([src, (ar * 97) % N])
    dstp = jnp.concatenate([dst, N + ar % (NP - N)])

    zrows = jnp.zeros((RPT, D), jnp.float32)
    zcnt = jnp.zeros((RPT,), jnp.float32)
    ones = jnp.ones((CH,), jnp.float32)
    xp = jnp.concatenate([x, jnp.zeros((NP - N, D), jnp.float32)])

    eidx = jnp.stack([srcp.reshape(-1, CH), dstp.reshape(-1, CH)], axis=1)

    sc_aggregate, sc_aggregate_cnt = _sc_kernels()

    h = xp
    cnt = None
    for (Wl, bl, Wr, relu, agg) in ((Wl1, bl1, Wr1, True, sc_aggregate_cnt),
                                    (Wl2, bl2, Wr2, True, sc_aggregate),
                                    (Wl3, bl3, Wr3, False, sc_aggregate)):
        p, cflat = agg(h, eidx, zrows, ones, zcnt)
        if cnt is None:
            cnt = cflat.reshape(NC, NP)
        h = _tc_sage_mm(relu, p, cnt, h, Wl.T, bl.reshape(1, D), Wr.T)
    return h[:N]


# speedup vs baseline: 13.0432x; 1.1149x over previous
import functools

import jax
import jax.numpy as jnp
from jax import lax
from jax.experimental import pallas as pl
from jax.experimental.pallas import tpu as pltpu
from jax.experimental.pallas import tpu_sc as plsc

N = 10000
E = 320000
D = 128

NC = 2
NS = 16
NW = NC * NS
CH = 128
EPT = 10240
EPAD = EPT * NW
NCHUNK = EPT // CH
NP = 10240
RPT = NP // NS
CW = 16

@functools.cache
def _sc_kernels():
    mesh = plsc.VectorSubcoreMesh(core_axis_name="c", subcore_axis_name="s",
                                  num_cores=NC, num_subcores=NS)

    def _agg_body(with_count, h_hbm, eidx_hbm, zrows_hbm, ones_hbm, zcnt_hbm,
                  out_hbm, cnt_out_hbm,
                  idx0, idx1, idx2, idx3, rows0, rows1,
                  ones_v, acc, cnt,
                  semg0, semg1, semi0, semi1, semi2, semi3, semc):
        cid = lax.axis_index("c")
        sid = lax.axis_index("s")
        wid = sid * NC + cid
        row0 = pl.multiple_of(sid * RPT, 8)
        pltpu.sync_copy(zrows_hbm, acc.at[pl.ds(row0, RPT)])
        if with_count:
            pltpu.sync_copy(zcnt_hbm, cnt.at[pl.ds(row0, RPT)])
            pltpu.sync_copy(ones_hbm, ones_v)
        plsc.subcore_barrier()
        base = wid * NCHUNK
        idx = (idx0, idx1, idx2, idx3)
        rows = (rows0, rows1)
        semg = (semg0, semg1)
        semi = (semi0, semi1, semi2, semi3)

        def load(g, b):
            pltpu.async_copy(eidx_hbm.at[base + g], idx[b], semi[b])

        def loadwait(b):
            pltpu.make_async_copy(eidx_hbm.at[base], idx[b], semi[b]).wait()

        def gather(b, r):
            pltpu.async_copy(h_hbm.at[idx[b].at[0]], rows[r], semg[r])

        def gatherwait(b, r):
            pltpu.make_async_copy(h_hbm.at[idx[b].at[0]], rows[r], semg[r]).wait()

        def scatter(b, r):
            if with_count:
                pltpu.async_copy(ones_v, cnt.at[idx[b].at[1]], semc, add=True)
            pltpu.sync_copy(rows[r], acc.at[idx[b].at[1]], add=True)
            if with_count:
                pltpu.make_async_copy(ones_v, cnt.at[idx[b].at[1]], semc).wait()

        load(0, 0)
        load(1, 1)
        load(2, 2)
        loadwait(0)
        gather(0, 0)

        def body(gg, carry):
            g = gg * 4
            loadwait(1); gather(1, 1); load(g + 3, 3)
            gatherwait(0, 0); scatter(0, 0); load(g + 4, 0)
            loadwait(2); gather(2, 0)
            gatherwait(1, 1); scatter(1, 1); load(g + 5, 1)
            loadwait(3); gather(3, 1)
            gatherwait(2, 0); scatter(2, 0); load(g + 6, 2)
            loadwait(0); gather(0, 0)
            gatherwait(3, 1); scatter(3, 1)
            return carry

        lax.fori_loop(0, NCHUNK // 4 - 1, body, 0)
        load(NCHUNK - 1, 3)
        loadwait(1); gather(1, 1)
        gatherwait(0, 0); scatter(0, 0)
        loadwait(2); gather(2, 0)
        gatherwait(1, 1); scatter(1, 1)
        loadwait(3); gather(3, 1)
        gatherwait(2, 0); scatter(2, 0)
        gatherwait(3, 1); scatter(3, 1)

        plsc.subcore_barrier()
        pltpu.sync_copy(acc.at[pl.ds(row0, RPT)],
                        out_hbm.at[cid, pl.ds(row0, RPT)])
        if with_count:
            cnt0 = pl.multiple_of(cid * NP + sid * RPT, 8)
            pltpu.sync_copy(cnt.at[pl.ds(row0, RPT)],
                            cnt_out_hbm.at[pl.ds(cnt0, RPT)])

    _agg_scratch = (
        [pltpu.VMEM((2, CH), jnp.int32) for _ in range(4)] +
        [pltpu.VMEM((CH, D), jnp.float32) for _ in range(2)] +
        [pltpu.VMEM((CH,), jnp.float32)] +
        [pltpu.VMEM_SHARED((NP, D), jnp.float32)] +
        [pltpu.VMEM_SHARED((NP,), jnp.float32)] +
        [pltpu.SemaphoreType.DMA for _ in range(7)]
    )

    @functools.partial(
        pl.kernel,
        out_type=(jax.ShapeDtypeStruct((NC, NP, D), jnp.float32),
                  jax.ShapeDtypeStruct((NC * NP,), jnp.float32)),
        mesh=mesh,
        scratch_types=_agg_scratch,
    )
    def sc_aggregate_cnt(*args):
        _agg_body(True, *args)

    @functools.partial(
        pl.kernel,
        out_type=(jax.ShapeDtypeStruct((NC, NP, D), jnp.float32),
                  jax.ShapeDtypeStruct((NC * NP,), jnp.float32)),
        mesh=mesh,
        scratch_types=_agg_scratch,
    )
    def sc_aggregate(*args):
        _agg_body(False, *args)

    return sc_aggregate, sc_aggregate_cnt


BN = 1024


def _mm_body(relu, p_ref, cnt_ref, h_ref, wlt_ref, bl_ref, wrt_ref, o_ref):
    cnt = cnt_ref[0] + cnt_ref[1]
    inv = 1.0 / jnp.maximum(cnt, 1.0)
    mean = (p_ref[0] + p_ref[1]) * inv[:, None]
    out = (jnp.dot(mean, wlt_ref[...], preferred_element_type=jnp.float32)
           + bl_ref[...]
           + jnp.dot(h_ref[...], wrt_ref[...], preferred_element_type=jnp.float32))
    if relu:
        out = jnp.maximum(out, 0.0)
    o_ref[...] = out


def _tc_sage_mm(relu, p, cnt, h, wlt, bl, wrt):
    return pl.pallas_call(
        functools.partial(_mm_body, relu),
        grid=(NP // BN,),
        in_specs=[
            pl.BlockSpec((NC, BN, D), lambda i: (0, i, 0)),
            pl.BlockSpec((NC, BN), lambda i: (0, i)),
            pl.BlockSpec((BN, D), lambda i: (i, 0)),
            pl.BlockSpec((D, D), lambda i: (0, 0)),
            pl.BlockSpec((1, D), lambda i: (0, 0)),
            pl.BlockSpec((D, D), lambda i: (0, 0)),
        ],
        out_specs=pl.BlockSpec((BN, D), lambda i: (i, 0)),
        out_shape=jax.ShapeDtypeStruct((NP, D), jnp.float32),
    )(p, cnt, h, wlt, bl, wrt)


def kernel(x, edge_index, Wl1, bl1, Wr1, Wl2, bl2, Wr2, Wl3, bl3, Wr3):
    src = edge_index[0].astype(jnp.int32)
    dst = edge_index[1].astype(jnp.int32)
    pad = EPAD - E
    ar = jnp.arange(pad, dtype=jnp.int32)
    srcp = jnp.concatenate([src, (ar * 97) % N])
    dstp = jnp.concatenate([dst, N + ar % (NP - N)])

    zrows = jnp.zeros((RPT, D), jnp.float32)
    zcnt = jnp.zeros((RPT,), jnp.float32)
    ones = jnp.ones((CH,), jnp.float32)
    xp = jnp.concatenate([x, jnp.zeros((NP - N, D), jnp.float32)])

    eidx = jnp.stack([srcp.reshape(-1, CH), dstp.reshape(-1, CH)], axis=1)

    sc_aggregate, sc_aggregate_cnt = _sc_kernels()

    h = xp
    cnt = None
    for (Wl, bl, Wr, relu, agg) in ((Wl1, bl1, Wr1, True, sc_aggregate_cnt),
                                    (Wl2, bl2, Wr2, True, sc_aggregate),
                                    (Wl3, bl3, Wr3, False, sc_aggregate)):
        p, cflat = agg(h, eidx, zrows, ones, zcnt)
        if cnt is None:
            cnt = cflat.reshape(NC, NP)
        h = _tc_sage_mm(relu, p, cnt, h, Wl.T, bl.reshape(1, D), Wr.T)
    return h[:N]


# speedup vs baseline: 13.5538x; 1.0391x over previous
import functools

import jax
import jax.numpy as jnp
from jax import lax
from jax.experimental import pallas as pl
from jax.experimental.pallas import tpu as pltpu
from jax.experimental.pallas import tpu_sc as plsc

N = 10000
E = 320000
D = 128

NC = 2
NS = 16
NW = NC * NS
CH = 80
EPT = 10240
EPAD = EPT * NW
NCHUNK = EPT // CH
NP = 10240
RPT = NP // NS
NIB = 8
NRB = 4
TOTCHUNK = EPAD // CH
EXTRA_CHUNKS = 8


@functools.cache
def _sc_kernels():
    mesh = plsc.VectorSubcoreMesh(core_axis_name="c", subcore_axis_name="s",
                                  num_cores=NC, num_subcores=NS)

    def _agg_body(with_count, h_hbm, eidx_hbm, zrows_hbm, ones_hbm, zcnt_hbm,
                  out_hbm, cnt_out_hbm, *scratch):
        idx = scratch[0:NIB]
        rows = scratch[NIB:NIB + NRB]
        ones_v = scratch[NIB + NRB]
        acc = scratch[NIB + NRB + 1]
        cnt = scratch[NIB + NRB + 2]
        semi = scratch[NIB + NRB + 3:NIB + NRB + 3 + NIB]
        semg = scratch[NIB + NRB + 3 + NIB:NIB + NRB + 3 + NIB + NRB]
        sems = scratch[NIB + NRB + 3 + NIB + NRB:NIB + NRB + 3 + NIB + 2 * NRB]
        semc = scratch[NIB + NRB + 3 + NIB + 2 * NRB]

        cid = lax.axis_index("c")
        sid = lax.axis_index("s")
        wid = sid * NC + cid
        row0 = pl.multiple_of(sid * RPT, 8)
        base = wid * NCHUNK

        def load(g, b):
            pltpu.async_copy(eidx_hbm.at[base + g], idx[b], semi[b])

        load(0, 0)
        load(1, 1)
        pltpu.sync_copy(zrows_hbm, acc.at[pl.ds(row0, RPT)])
        if with_count:
            pltpu.sync_copy(zcnt_hbm, cnt.at[pl.ds(row0, RPT)])
            pltpu.sync_copy(ones_hbm, ones_v)
        plsc.subcore_barrier()

        def loadwait(b):
            pltpu.make_async_copy(eidx_hbm.at[base], idx[b], semi[b]).wait()

        def gather(b, r):
            pltpu.async_copy(h_hbm.at[idx[b].at[0]], rows[r], semg[r])

        def gatherwait(b, r):
            pltpu.make_async_copy(h_hbm.at[idx[b].at[0]], rows[r], semg[r]).wait()

        def scatter(b, r):
            if with_count:
                pltpu.async_copy(ones_v, cnt.at[idx[b].at[1]], semc, add=True)
            pltpu.async_copy(rows[r], acc.at[idx[b].at[1]], sems[r], add=True)

        def scatwait(b, r):
            pltpu.make_async_copy(rows[r], acc.at[idx[b].at[1]], sems[r]).wait()
            if with_count:
                pltpu.make_async_copy(ones_v, cnt.at[idx[b].at[1]], semc).wait()

        def block(g):
            if g >= 4:
                scatwait((g - 4) % NIB, (g - 4) % NRB)
            load(g + 2, (g + 2) % NIB)
            loadwait(g % NIB)
            gather(g % NIB, g % NRB)
            if g >= 2:
                gatherwait((g - 2) % NIB, (g - 2) % NRB)
                scatter((g - 2) % NIB, (g - 2) % NRB)

        for g in range(NIB):
            block(g)

        def body(gg, carry):
            g0 = gg * NIB
            for j in range(NIB):
                g = g0 + j
                scatwait((j - 4) % NIB, (j - 4) % NRB)
                load(g + 2, (j + 2) % NIB)
                loadwait(j % NIB)
                gather(j % NIB, j % NRB)
                gatherwait((j - 2) % NIB, (j - 2) % NRB)
                scatter((j - 2) % NIB, (j - 2) % NRB)
            return carry

        lax.fori_loop(1, NCHUNK // NIB, body, 0)
        gatherwait((NCHUNK - 2) % NIB, (NCHUNK - 2) % NRB)
        scatter((NCHUNK - 2) % NIB, (NCHUNK - 2) % NRB)
        gatherwait((NCHUNK - 1) % NIB, (NCHUNK - 1) % NRB)
        scatter((NCHUNK - 1) % NIB, (NCHUNK - 1) % NRB)
        for g in range(NCHUNK - 4, NCHUNK):
            scatwait(g % NIB, g % NRB)
        loadwait(NCHUNK % NIB)
        loadwait((NCHUNK + 1) % NIB)

        plsc.subcore_barrier()
        pltpu.sync_copy(acc.at[pl.ds(row0, RPT)],
                        out_hbm.at[cid, pl.ds(row0, RPT)])
        if with_count:
            cnt0 = pl.multiple_of(cid * NP + sid * RPT, 8)
            pltpu.sync_copy(cnt.at[pl.ds(row0, RPT)],
                            cnt_out_hbm.at[pl.ds(cnt0, RPT)])

    _agg_scratch = (
        [pltpu.VMEM((2, CH), jnp.int32) for _ in range(NIB)] +
        [pltpu.VMEM((CH, D), jnp.float32) for _ in range(NRB)] +
        [pltpu.VMEM((CH,), jnp.float32)] +
        [pltpu.VMEM_SHARED((NP, D), jnp.float32)] +
        [pltpu.VMEM_SHARED((NP,), jnp.float32)] +
        [pltpu.SemaphoreType.DMA for _ in range(NIB + 2 * NRB + 1)]
    )

    @functools.partial(
        pl.kernel,
        out_type=(jax.ShapeDtypeStruct((NC, NP, D), jnp.float32),
                  jax.ShapeDtypeStruct((NC * NP,), jnp.float32)),
        mesh=mesh,
        scratch_types=_agg_scratch,
    )
    def sc_aggregate_cnt(*args):
        _agg_body(True, *args)

    @functools.partial(
        pl.kernel,
        out_type=(jax.ShapeDtypeStruct((NC, NP, D), jnp.float32),
                  jax.ShapeDtypeStruct((NC * NP,), jnp.float32)),
        mesh=mesh,
        scratch_types=_agg_scratch,
    )
    def sc_aggregate(*args):
        _agg_body(False, *args)

    return sc_aggregate, sc_aggregate_cnt


BN = 1024


def _mm_body(relu, p_ref, cnt_ref, h_ref, wlt_ref, bl_ref, wrt_ref, o_ref):
    cnt = cnt_ref[0] + cnt_ref[1]
    inv = 1.0 / jnp.maximum(cnt, 1.0)
    mean = (p_ref[0] + p_ref[1]) * inv[:, None]
    out = (jnp.dot(mean, wlt_ref[...], preferred_element_type=jnp.float32)
           + bl_ref[...]
           + jnp.dot(h_ref[...], wrt_ref[...], preferred_element_type=jnp.float32))
    if relu:
        out = jnp.maximum(out, 0.0)
    o_ref[...] = out


def _tc_sage_mm(relu, p, cnt, h, wlt, bl, wrt):
    return pl.pallas_call(
        functools.partial(_mm_body, relu),
        grid=(NP // BN,),
        in_specs=[
            pl.BlockSpec((NC, BN, D), lambda i: (0, i, 0)),
            pl.BlockSpec((NC, BN), lambda i: (0, i)),
            pl.BlockSpec((BN, D), lambda i: (i, 0)),
            pl.BlockSpec((D, D), lambda i: (0, 0)),
            pl.BlockSpec((1, D), lambda i: (0, 0)),
            pl.BlockSpec((D, D), lambda i: (0, 0)),
        ],
        out_specs=pl.BlockSpec((BN, D), lambda i: (i, 0)),
        out_shape=jax.ShapeDtypeStruct((NP, D), jnp.float32),
    )(p, cnt, h, wlt, bl, wrt)


def kernel(x, edge_index, Wl1, bl1, Wr1, Wl2, bl2, Wr2, Wl3, bl3, Wr3):
    src = edge_index[0].astype(jnp.int32)
    dst = edge_index[1].astype(jnp.int32)
    pad = EPAD - E
    ar = jnp.arange(pad, dtype=jnp.int32)
    srcp = jnp.concatenate([src, (ar * 97) % N])
    dstp = jnp.concatenate([dst, N + ar % (NP - N)])

    zrows = jnp.zeros((RPT, D), jnp.float32)
    zcnt = jnp.zeros((RPT,), jnp.float32)
    ones = jnp.ones((CH,), jnp.float32)
    xp = jnp.concatenate([x, jnp.zeros((NP - N, D), jnp.float32)])

    eidx = jnp.concatenate([
        jnp.stack([srcp.reshape(-1, CH), dstp.reshape(-1, CH)], axis=1),
        jnp.zeros((EXTRA_CHUNKS, 2, CH), jnp.int32),
    ])

    sc_aggregate, sc_aggregate_cnt = _sc_kernels()

    h = xp
    cnt = None
    for (Wl, bl, Wr, relu, agg) in ((Wl1, bl1, Wr1, True, sc_aggregate_cnt),
                                    (Wl2, bl2, Wr2, True, sc_aggregate),
                                    (Wl3, bl3, Wr3, False, sc_aggregate)):
        p, cflat = agg(h, eidx, zrows, ones, zcnt)
        if cnt is None:
            cnt = cflat.reshape(NC, NP)
        h = _tc_sage_mm(relu, p, cnt, h, Wl.T, bl.reshape(1, D), Wr.T)
    return h[:N]


# speedup vs baseline: 13.7208x; 1.0123x over previous
import functools

import jax
import jax.numpy as jnp
from jax import lax
from jax.experimental import pallas as pl
from jax.experimental.pallas import tpu as pltpu
from jax.experimental.pallas import tpu_sc as plsc

N = 10000
E = 320000
D = 128

NC = 2
NS = 16
NW = NC * NS
CH = 80
EPT = E // NW
NCHUNK = EPT // CH
NP = 10240
RPT = NP // NS
NIB = 8
NRB = 4
TOTCHUNK = E // CH
EXTRA_CHUNKS = 8


@functools.cache
def _sc_kernels():
    mesh = plsc.VectorSubcoreMesh(core_axis_name="c", subcore_axis_name="s",
                                  num_cores=NC, num_subcores=NS)

    def _agg_body(with_count, h_hbm, eidx_hbm, zrows_hbm, ones_hbm, zcnt_hbm,
                  out_hbm, cnt_out_hbm, *scratch):
        idx = scratch[0:NIB]
        rows = scratch[NIB:NIB + NRB]
        ones_v = scratch[NIB + NRB]
        acc = scratch[NIB + NRB + 1]
        cnt = scratch[NIB + NRB + 2]
        semi = scratch[NIB + NRB + 3:NIB + NRB + 3 + NIB]
        semg = scratch[NIB + NRB + 3 + NIB:NIB + NRB + 3 + NIB + NRB]
        sems = scratch[NIB + NRB + 3 + NIB + NRB:NIB + NRB + 3 + NIB + 2 * NRB]
        semc = scratch[NIB + NRB + 3 + NIB + 2 * NRB]

        cid = lax.axis_index("c")
        sid = lax.axis_index("s")
        wid = sid * NC + cid
        row0 = pl.multiple_of(sid * RPT, 8)
        base = wid * NCHUNK

        def load(g, b):
            pltpu.async_copy(eidx_hbm.at[base + g], idx[b], semi[b])

        load(0, 0)
        load(1, 1)
        pltpu.sync_copy(zrows_hbm, acc.at[pl.ds(row0, RPT)])
        if with_count:
            pltpu.sync_copy(zcnt_hbm, cnt.at[pl.ds(row0, RPT)])
            pltpu.sync_copy(ones_hbm, ones_v)
        plsc.subcore_barrier()

        def loadwait(b):
            pltpu.make_async_copy(eidx_hbm.at[base], idx[b], semi[b]).wait()

        def gather(b, r):
            pltpu.async_copy(h_hbm.at[idx[b].at[0]], rows[r], semg[r])

        def gatherwait(b, r):
            pltpu.make_async_copy(h_hbm.at[idx[b].at[0]], rows[r], semg[r]).wait()

        def scatter(b, r):
            if with_count:
                pltpu.async_copy(ones_v, cnt.at[idx[b].at[1]], semc, add=True)
            pltpu.async_copy(rows[r], acc.at[idx[b].at[1]], sems[r], add=True)

        def scatwait(b, r):
            pltpu.make_async_copy(rows[r], acc.at[idx[b].at[1]], sems[r]).wait()
            if with_count:
                pltpu.make_async_copy(ones_v, cnt.at[idx[b].at[1]], semc).wait()

        def block(g):
            if g >= 4:
                scatwait((g - 4) % NIB, (g - 4) % NRB)
            load(g + 2, (g + 2) % NIB)
            loadwait(g % NIB)
            gather(g % NIB, g % NRB)
            if g >= 2:
                gatherwait((g - 2) % NIB, (g - 2) % NRB)
                scatter((g - 2) % NIB, (g - 2) % NRB)

        for g in range(NIB):
            block(g)

        def body(gg, carry):
            g0 = gg * NIB
            for j in range(NIB):
                g = g0 + j
                scatwait((j - 4) % NIB, (j - 4) % NRB)
                load(g + 2, (j + 2) % NIB)
                loadwait(j % NIB)
                gather(j % NIB, j % NRB)
                gatherwait((j - 2) % NIB, (j - 2) % NRB)
                scatter((j - 2) % NIB, (j - 2) % NRB)
            return carry

        lax.fori_loop(1, NCHUNK // NIB, body, 0)
        for g in range((NCHUNK // NIB) * NIB, NCHUNK):
            block(g)
        gatherwait((NCHUNK - 2) % NIB, (NCHUNK - 2) % NRB)
        scatter((NCHUNK - 2) % NIB, (NCHUNK - 2) % NRB)
        gatherwait((NCHUNK - 1) % NIB, (NCHUNK - 1) % NRB)
        scatter((NCHUNK - 1) % NIB, (NCHUNK - 1) % NRB)
        for g in range(NCHUNK - 4, NCHUNK):
            scatwait(g % NIB, g % NRB)
        loadwait(NCHUNK % NIB)
        loadwait((NCHUNK + 1) % NIB)

        plsc.subcore_barrier()
        pltpu.sync_copy(acc.at[pl.ds(row0, RPT)],
                        out_hbm.at[cid, pl.ds(row0, RPT)])
        if with_count:
            cnt0 = pl.multiple_of(cid * NP + sid * RPT, 8)
            pltpu.sync_copy(cnt.at[pl.ds(row0, RPT)],
                            cnt_out_hbm.at[pl.ds(cnt0, RPT)])

    _agg_scratch = (
        [pltpu.VMEM((2, CH), jnp.int32) for _ in range(NIB)] +
        [pltpu.VMEM((CH, D), jnp.float32) for _ in range(NRB)] +
        [pltpu.VMEM((CH,), jnp.float32)] +
        [pltpu.VMEM_SHARED((NP, D), jnp.float32)] +
        [pltpu.VMEM_SHARED((NP,), jnp.float32)] +
        [pltpu.SemaphoreType.DMA for _ in range(NIB + 2 * NRB + 1)]
    )

    @functools.partial(
        pl.kernel,
        out_type=(jax.ShapeDtypeStruct((NC, NP, D), jnp.float32),
                  jax.ShapeDtypeStruct((NC * NP,), jnp.float32)),
        mesh=mesh,
        scratch_types=_agg_scratch,
    )
    def sc_aggregate_cnt(*args):
        _agg_body(True, *args)

    @functools.partial(
        pl.kernel,
        out_type=(jax.ShapeDtypeStruct((NC, NP, D), jnp.float32),
                  jax.ShapeDtypeStruct((NC * NP,), jnp.float32)),
        mesh=mesh,
        scratch_types=_agg_scratch,
    )
    def sc_aggregate(*args):
        _agg_body(False, *args)

    return sc_aggregate, sc_aggregate_cnt


BN = 1024


def _mm_body(relu, p_ref, cnt_ref, h_ref, wlt_ref, bl_ref, wrt_ref, o_ref):
    cnt = cnt_ref[0] + cnt_ref[1]
    inv = 1.0 / jnp.maximum(cnt, 1.0)
    mean = (p_ref[0] + p_ref[1]) * inv[:, None]
    out = (jnp.dot(mean, wlt_ref[...], preferred_element_type=jnp.float32)
           + bl_ref[...]
           + jnp.dot(h_ref[...], wrt_ref[...], preferred_element_type=jnp.float32))
    if relu:
        out = jnp.maximum(out, 0.0)
    o_ref[...] = out


def _tc_sage_mm(relu, p, cnt, h, wlt, bl, wrt):
    return pl.pallas_call(
        functools.partial(_mm_body, relu),
        grid=(NP // BN,),
        in_specs=[
            pl.BlockSpec((NC, BN, D), lambda i: (0, i, 0)),
            pl.BlockSpec((NC, BN), lambda i: (0, i)),
            pl.BlockSpec((BN, D), lambda i: (i, 0)),
            pl.BlockSpec((D, D), lambda i: (0, 0)),
            pl.BlockSpec((1, D), lambda i: (0, 0)),
            pl.BlockSpec((D, D), lambda i: (0, 0)),
        ],
        out_specs=pl.BlockSpec((BN, D), lambda i: (i, 0)),
        out_shape=jax.ShapeDtypeStruct((NP, D), jnp.float32),
    )(p, cnt, h, wlt, bl, wrt)


def kernel(x, edge_index, Wl1, bl1, Wr1, Wl2, bl2, Wr2, Wl3, bl3, Wr3):
    ei = edge_index.astype(jnp.int32)

    zrows = jnp.zeros((RPT, D), jnp.float32)
    zcnt = jnp.zeros((RPT,), jnp.float32)
    ones = jnp.ones((CH,), jnp.float32)
    xp = jnp.concatenate([x, jnp.zeros((NP - N, D), jnp.float32)])

    eidx = jnp.concatenate([
        jnp.stack([ei[0].reshape(-1, CH), ei[1].reshape(-1, CH)], axis=1),
        jnp.zeros((EXTRA_CHUNKS, 2, CH), jnp.int32),
    ])

    sc_aggregate, sc_aggregate_cnt = _sc_kernels()

    h = xp
    cnt = None
    for (Wl, bl, Wr, relu, agg) in ((Wl1, bl1, Wr1, True, sc_aggregate_cnt),
                                    (Wl2, bl2, Wr2, True, sc_aggregate),
                                    (Wl3, bl3, Wr3, False, sc_aggregate)):
        p, cflat = agg(h, eidx, zrows, ones, zcnt)
        if cnt is None:
            cnt = cflat.reshape(NC, NP)
        h = _tc_sage_mm(relu, p, cnt, h, Wl.T, bl.reshape(1, D), Wr.T)
    return h[:N]


# speedup vs baseline: 13.9730x; 1.0184x over previous
import functools

import jax
import jax.numpy as jnp
from jax import lax
from jax.experimental import pallas as pl
from jax.experimental.pallas import tpu as pltpu
from jax.experimental.pallas import tpu_sc as plsc

N = 10000
E = 320000
D = 128

NC = 2
NS = 16
NW = NC * NS
CH = 80
EPT = E // NW
NCHUNK = EPT // CH
NP = 10240
RPT = NP // NS
NIB = 8
NRB = 4
TOTCHUNK = E // CH
EXTRA_CHUNKS = 8


@functools.cache
def _sc_kernels():
    mesh = plsc.VectorSubcoreMesh(core_axis_name="c", subcore_axis_name="s",
                                  num_cores=NC, num_subcores=NS)

    def _agg_body(with_count, h_hbm, eidx_hbm, zrows_hbm, ones_hbm, zcnt_hbm,
                  out_hbm, cnt_out_hbm, *scratch):
        idx = scratch[0:NIB]
        rows = scratch[NIB:NIB + NRB]
        ones_v = scratch[NIB + NRB]
        acc = scratch[NIB + NRB + 1]
        cnt = scratch[NIB + NRB + 2]
        semi = scratch[NIB + NRB + 3:NIB + NRB + 3 + NIB]
        semg = scratch[NIB + NRB + 3 + NIB:NIB + NRB + 3 + NIB + NRB]
        sems = scratch[NIB + NRB + 3 + NIB + NRB:NIB + NRB + 3 + NIB + 2 * NRB]
        semc = scratch[NIB + NRB + 3 + NIB + 2 * NRB]

        cid = lax.axis_index("c")
        sid = lax.axis_index("s")
        wid = sid * NC + cid
        row0 = pl.multiple_of(sid * RPT, 8)
        base = wid * NCHUNK

        def load(g, b):
            pltpu.async_copy(eidx_hbm.at[base + g], idx[b], semi[b])

        load(0, 0)
        load(1, 1)
        pltpu.sync_copy(zrows_hbm, acc.at[pl.ds(row0, RPT)])
        if with_count:
            pltpu.sync_copy(zcnt_hbm, cnt.at[pl.ds(row0, RPT)])
            pltpu.sync_copy(ones_hbm, ones_v)
        plsc.subcore_barrier()

        def loadwait(b):
            pltpu.make_async_copy(eidx_hbm.at[base], idx[b], semi[b]).wait()

        def gather(b, r):
            pltpu.async_copy(h_hbm.at[idx[b].at[0]], rows[r], semg[r])

        def gatherwait(b, r):
            pltpu.make_async_copy(h_hbm.at[idx[b].at[0]], rows[r], semg[r]).wait()

        def scatter(b, r):
            if with_count:
                pltpu.async_copy(ones_v, cnt.at[idx[b].at[1]], semc, add=True)
            pltpu.async_copy(rows[r], acc.at[idx[b].at[1]], sems[r], add=True)

        def scatwait(b, r):
            pltpu.make_async_copy(rows[r], acc.at[idx[b].at[1]], sems[r]).wait()
            if with_count:
                pltpu.make_async_copy(ones_v, cnt.at[idx[b].at[1]], semc).wait()

        def block(g):
            if g >= 4:
                scatwait((g - 4) % NIB, (g - 4) % NRB)
            load(g + 2, (g + 2) % NIB)
            loadwait(g % NIB)
            gather(g % NIB, g % NRB)
            if g >= 2:
                gatherwait((g - 2) % NIB, (g - 2) % NRB)
                scatter((g - 2) % NIB, (g - 2) % NRB)

        for g in range(NIB):
            block(g)

        def body(gg, carry):
            g0 = gg * NIB
            for j in range(NIB):
                g = g0 + j
                scatwait((j - 4) % NIB, (j - 4) % NRB)
                load(g + 2, (j + 2) % NIB)
                loadwait(j % NIB)
                gather(j % NIB, j % NRB)
                gatherwait((j - 2) % NIB, (j - 2) % NRB)
                scatter((j - 2) % NIB, (j - 2) % NRB)
            return carry

        lax.fori_loop(1, NCHUNK // NIB, body, 0)
        for g in range((NCHUNK // NIB) * NIB, NCHUNK):
            block(g)
        gatherwait((NCHUNK - 2) % NIB, (NCHUNK - 2) % NRB)
        scatter((NCHUNK - 2) % NIB, (NCHUNK - 2) % NRB)
        gatherwait((NCHUNK - 1) % NIB, (NCHUNK - 1) % NRB)
        scatter((NCHUNK - 1) % NIB, (NCHUNK - 1) % NRB)
        for g in range(NCHUNK - 4, NCHUNK):
            scatwait(g % NIB, g % NRB)
        loadwait(NCHUNK % NIB)
        loadwait((NCHUNK + 1) % NIB)

        plsc.subcore_barrier()
        pltpu.sync_copy(acc.at[pl.ds(row0, RPT)],
                        out_hbm.at[cid, pl.ds(row0, RPT)])
        if with_count:
            cnt0 = pl.multiple_of(cid * NP + sid * RPT, 8)
            pltpu.sync_copy(cnt.at[pl.ds(row0, RPT)],
                            cnt_out_hbm.at[pl.ds(cnt0, RPT)])

    _agg_scratch = (
        [pltpu.VMEM((2, CH), jnp.int32) for _ in range(NIB)] +
        [pltpu.VMEM((CH, D), jnp.float32) for _ in range(NRB)] +
        [pltpu.VMEM((CH,), jnp.float32)] +
        [pltpu.VMEM_SHARED((NP, D), jnp.float32)] +
        [pltpu.VMEM_SHARED((NP,), jnp.float32)] +
        [pltpu.SemaphoreType.DMA for _ in range(NIB + 2 * NRB + 1)]
    )

    @functools.partial(
        pl.kernel,
        out_type=(jax.ShapeDtypeStruct((NC, NP, D), jnp.float32),
                  jax.ShapeDtypeStruct((NC * NP,), jnp.float32)),
        mesh=mesh,
        scratch_types=_agg_scratch,
    )
    def sc_aggregate_cnt(*args):
        _agg_body(True, *args)

    @functools.partial(
        pl.kernel,
        out_type=(jax.ShapeDtypeStruct((NC, NP, D), jnp.float32),
                  jax.ShapeDtypeStruct((NC * NP,), jnp.float32)),
        mesh=mesh,
        scratch_types=_agg_scratch,
    )
    def sc_aggregate(*args):
        _agg_body(False, *args)

    return sc_aggregate, sc_aggregate_cnt


BN = 2048


def _mm_body(relu, p_ref, cnt_ref, h_ref, wlt_ref, bl_ref, wrt_ref, o_ref):
    cnt = cnt_ref[0] + cnt_ref[1]
    inv = 1.0 / jnp.maximum(cnt, 1.0)
    mean = (p_ref[0] + p_ref[1]) * inv[:, None]
    out = (jnp.dot(mean, wlt_ref[...], preferred_element_type=jnp.float32)
           + bl_ref[...]
           + jnp.dot(h_ref[...], wrt_ref[...], preferred_element_type=jnp.float32))
    if relu:
        out = jnp.maximum(out, 0.0)
    o_ref[...] = out


def _tc_sage_mm(relu, p, cnt, h, wlt, bl, wrt):
    return pl.pallas_call(
        functools.partial(_mm_body, relu),
        grid=(NP // BN,),
        in_specs=[
            pl.BlockSpec((NC, BN, D), lambda i: (0, i, 0)),
            pl.BlockSpec((NC, BN), lambda i: (0, i)),
            pl.BlockSpec((BN, D), lambda i: (i, 0)),
            pl.BlockSpec((D, D), lambda i: (0, 0)),
            pl.BlockSpec((1, D), lambda i: (0, 0)),
            pl.BlockSpec((D, D), lambda i: (0, 0)),
        ],
        out_specs=pl.BlockSpec((BN, D), lambda i: (i, 0)),
        out_shape=jax.ShapeDtypeStruct((NP, D), jnp.float32),
    )(p, cnt, h, wlt, bl, wrt)


def kernel(x, edge_index, Wl1, bl1, Wr1, Wl2, bl2, Wr2, Wl3, bl3, Wr3):
    ei = edge_index.astype(jnp.int32)

    zrows = jnp.zeros((RPT, D), jnp.float32)
    zcnt = jnp.zeros((RPT,), jnp.float32)
    ones = jnp.ones((CH,), jnp.float32)
    xp = jnp.concatenate([x, jnp.zeros((NP - N, D), jnp.float32)])

    eidx = jnp.concatenate([
        jnp.stack([ei[0].reshape(-1, CH), ei[1].reshape(-1, CH)], axis=1),
        jnp.zeros((EXTRA_CHUNKS, 2, CH), jnp.int32),
    ])

    sc_aggregate, sc_aggregate_cnt = _sc_kernels()

    h = xp
    cnt = None
    for (Wl, bl, Wr, relu, agg) in ((Wl1, bl1, Wr1, True, sc_aggregate_cnt),
                                    (Wl2, bl2, Wr2, True, sc_aggregate),
                                    (Wl3, bl3, Wr3, False, sc_aggregate)):
        p, cflat = agg(h, eidx, zrows, ones, zcnt)
        if cnt is None:
            cnt = cflat.reshape(NC, NP)
        h = _tc_sage_mm(relu, p, cnt, h, Wl.T, bl.reshape(1, D), Wr.T)
    return h[:N]


# speedup vs baseline: 14.3951x; 1.0302x over previous
import functools

import jax
import jax.numpy as jnp
from jax import lax
from jax.experimental import pallas as pl
from jax.experimental.pallas import tpu as pltpu
from jax.experimental.pallas import tpu_sc as plsc

N = 10000
E = 320000
D = 128

NC = 2
NS = 16
NW = NC * NS
CH = 80
EPT = E // NW
NCHUNK = EPT // CH
NP = 10240
RPT = NP // NS
NIB = 8
NRB = 4
TOTCHUNK = E // CH
EXTRA_CHUNKS = 8


@functools.cache
def _sc_kernels():
    mesh = plsc.VectorSubcoreMesh(core_axis_name="c", subcore_axis_name="s",
                                  num_cores=NC, num_subcores=NS)

    def _agg_body(with_count, h_hbm, eidx_hbm, zrows_hbm, ones_hbm, zcnt_hbm,
                  out_hbm, cnt_out_hbm, *scratch):
        idx = scratch[0:NIB]
        rows = scratch[NIB:NIB + NRB]
        ones_v = scratch[NIB + NRB]
        acc = scratch[NIB + NRB + 1]
        cnt = scratch[NIB + NRB + 2]
        semi = scratch[NIB + NRB + 3:NIB + NRB + 3 + NIB]
        semg = scratch[NIB + NRB + 3 + NIB:NIB + NRB + 3 + NIB + NRB]
        sems = scratch[NIB + NRB + 3 + NIB + NRB:NIB + NRB + 3 + NIB + 2 * NRB]
        semc = scratch[NIB + NRB + 3 + NIB + 2 * NRB]

        cid = lax.axis_index("c")
        sid = lax.axis_index("s")
        wid = sid * NC + cid
        row0 = pl.multiple_of(sid * RPT, 8)
        base = wid * NCHUNK

        def load(g, b):
            pltpu.async_copy(eidx_hbm.at[base + g], idx[b], semi[b])

        load(0, 0)
        load(1, 1)
        pltpu.sync_copy(zrows_hbm, acc.at[pl.ds(row0, RPT)])
        if with_count:
            pltpu.sync_copy(zcnt_hbm, cnt.at[pl.ds(row0, RPT)])
            pltpu.sync_copy(ones_hbm, ones_v)
        plsc.subcore_barrier()

        def loadwait(b):
            pltpu.make_async_copy(eidx_hbm.at[base], idx[b], semi[b]).wait()

        def gather(b, r):
            pltpu.async_copy(h_hbm.at[idx[b].at[0]], rows[r], semg[r])

        def gatherwait(b, r):
            pltpu.make_async_copy(h_hbm.at[idx[b].at[0]], rows[r], semg[r]).wait()

        def scatter(b, r):
            if with_count:
                pltpu.async_copy(ones_v, cnt.at[idx[b].at[1]], semc, add=True)
            pltpu.async_copy(rows[r], acc.at[idx[b].at[1]], sems[r], add=True)

        def scatwait(b, r):
            pltpu.make_async_copy(rows[r], acc.at[idx[b].at[1]], sems[r]).wait()
            if with_count:
                pltpu.make_async_copy(ones_v, cnt.at[idx[b].at[1]], semc).wait()

        def block(g):
            if g >= 4:
                scatwait((g - 4) % NIB, (g - 4) % NRB)
            load(g + 2, (g + 2) % NIB)
            loadwait(g % NIB)
            gather(g % NIB, g % NRB)
            if g >= 2:
                gatherwait((g - 2) % NIB, (g - 2) % NRB)
                scatter((g - 2) % NIB, (g - 2) % NRB)

        for g in range(NIB):
            block(g)

        def body(gg, carry):
            g0 = gg * NIB
            for j in range(NIB):
                g = g0 + j
                scatwait((j - 4) % NIB, (j - 4) % NRB)
                load(g + 2, (j + 2) % NIB)
                loadwait(j % NIB)
                gather(j % NIB, j % NRB)
                gatherwait((j - 2) % NIB, (j - 2) % NRB)
                scatter((j - 2) % NIB, (j - 2) % NRB)
            return carry

        lax.fori_loop(1, NCHUNK // NIB, body, 0)
        for g in range((NCHUNK // NIB) * NIB, NCHUNK):
            block(g)
        gatherwait((NCHUNK - 2) % NIB, (NCHUNK - 2) % NRB)
        scatter((NCHUNK - 2) % NIB, (NCHUNK - 2) % NRB)
        gatherwait((NCHUNK - 1) % NIB, (NCHUNK - 1) % NRB)
        scatter((NCHUNK - 1) % NIB, (NCHUNK - 1) % NRB)
        for g in range(NCHUNK - 4, NCHUNK):
            scatwait(g % NIB, g % NRB)
        loadwait(NCHUNK % NIB)
        loadwait((NCHUNK + 1) % NIB)

        plsc.subcore_barrier()
        pltpu.sync_copy(acc.at[pl.ds(row0, RPT)],
                        out_hbm.at[cid, pl.ds(row0, RPT)])
        if with_count:
            cnt0 = pl.multiple_of(cid * NP + sid * RPT, 8)
            pltpu.sync_copy(cnt.at[pl.ds(row0, RPT)],
                            cnt_out_hbm.at[pl.ds(cnt0, RPT)])

    _agg_scratch = (
        [pltpu.VMEM((2, CH), jnp.int32) for _ in range(NIB)] +
        [pltpu.VMEM((CH, D), jnp.float32) for _ in range(NRB)] +
        [pltpu.VMEM((CH,), jnp.float32)] +
        [pltpu.VMEM_SHARED((NP, D), jnp.float32)] +
        [pltpu.VMEM_SHARED((NP,), jnp.float32)] +
        [pltpu.SemaphoreType.DMA for _ in range(NIB + 2 * NRB + 1)]
    )

    @functools.partial(
        pl.kernel,
        out_type=(jax.ShapeDtypeStruct((NC, NP, D), jnp.float32),
                  jax.ShapeDtypeStruct((NC * NP,), jnp.float32)),
        mesh=mesh,
        scratch_types=_agg_scratch,
    )
    def sc_aggregate_cnt(*args):
        _agg_body(True, *args)

    @functools.partial(
        pl.kernel,
        out_type=(jax.ShapeDtypeStruct((NC, NP, D), jnp.float32),
                  jax.ShapeDtypeStruct((NC * NP,), jnp.float32)),
        mesh=mesh,
        scratch_types=_agg_scratch,
    )
    def sc_aggregate(*args):
        _agg_body(False, *args)

    return sc_aggregate, sc_aggregate_cnt


BN = 2048


def _mm_body(relu, p_ref, cnt_ref, h_ref, wlt_ref, bl_ref, wrt_ref, o_ref):
    cnt = cnt_ref[0] + cnt_ref[1]
    inv = 1.0 / jnp.maximum(cnt, 1.0)
    mean = (p_ref[0] + p_ref[1]) * inv[:, None]
    out = (jnp.dot(mean, wlt_ref[...], preferred_element_type=jnp.float32)
           + bl_ref[...]
           + jnp.dot(h_ref[...], wrt_ref[...], preferred_element_type=jnp.float32))
    if relu:
        out = jnp.maximum(out, 0.0)
    o_ref[...] = out


def _tc_sage_mm(relu, p, cnt, h, wlt, bl, wrt):
    return pl.pallas_call(
        functools.partial(_mm_body, relu),
        grid=(NP // BN,),
        in_specs=[
            pl.BlockSpec((NC, BN, D), lambda i: (0, i, 0)),
            pl.BlockSpec((NC, BN), lambda i: (0, i)),
            pl.BlockSpec((BN, D), lambda i: (i, 0)),
            pl.BlockSpec((D, D), lambda i: (0, 0)),
            pl.BlockSpec((1, D), lambda i: (0, 0)),
            pl.BlockSpec((D, D), lambda i: (0, 0)),
        ],
        out_specs=pl.BlockSpec((BN, D), lambda i: (i, 0)),
        out_shape=jax.ShapeDtypeStruct((N, D), jnp.float32),
    )(p, cnt, h, wlt, bl, wrt)


def kernel(x, edge_index, Wl1, bl1, Wr1, Wl2, bl2, Wr2, Wl3, bl3, Wr3):
    ei = edge_index.astype(jnp.int32)

    zrows = jnp.zeros((RPT, D), jnp.float32)
    zcnt = jnp.zeros((RPT,), jnp.float32)
    ones = jnp.ones((CH,), jnp.float32)

    eidx = jnp.concatenate([
        jnp.stack([ei[0].reshape(-1, CH), ei[1].reshape(-1, CH)], axis=1),
        jnp.zeros((EXTRA_CHUNKS, 2, CH), jnp.int32),
    ])

    sc_aggregate, sc_aggregate_cnt = _sc_kernels()

    h = x
    cnt = None
    for (Wl, bl, Wr, relu, agg) in ((Wl1, bl1, Wr1, True, sc_aggregate_cnt),
                                    (Wl2, bl2, Wr2, True, sc_aggregate),
                                    (Wl3, bl3, Wr3, False, sc_aggregate)):
        p, cflat = agg(h, eidx, zrows, ones, zcnt)
        if cnt is None:
            cnt = cflat.reshape(NC, NP)
        h = _tc_sage_mm(relu, p, cnt, h, Wl.T, bl.reshape(1, D), Wr.T)
    return h
